# Initial kernel scaffold; baseline (speedup 1.0000x reference)
import jax
import jax.numpy as jnp
from jax.experimental import pallas as pl


def kernel(x_nchw, w1, b1, gamma, beta, w2, b2):
    raise NotImplementedError("write your pallas kernel here")


# speedup vs baseline: 1.3433x; 1.3433x over previous
import functools

import jax
import jax.numpy as jnp
from jax.experimental import pallas as pl
from jax.experimental.pallas import tpu as pltpu

_BN_EPS = 1e-5


def _fused_kernel(x_ref, w1_ref, gamma_ref, beta_ref, w2_ref, b2_ref,
                  o_ref, w1s_ref, shift_ref, *, n_batch, apply_block):
    s = pl.program_id(0)

    @pl.when(s == 0)
    def _stats_and_fold():
        cin = x_ref.shape[1]
        colsum = jnp.zeros((cin, 1), jnp.float32)
        gram = jnp.zeros((cin, cin), jnp.float32)
        for n in range(n_batch):
            xn = x_ref[n]
            colsum += jnp.sum(xn, axis=1, keepdims=True)
            gram += jax.lax.dot_general(
                xn, xn, (((1,), (1,)), ((), ())),
                preferred_element_type=jnp.float32)
        sum_h = jax.lax.dot_general(
            w1_ref[...], colsum, (((1,), (0,)), ((), ())),
            preferred_element_type=jnp.float32,
            precision=jax.lax.Precision.HIGHEST)
        wg = jax.lax.dot_general(
            w1_ref[...], gram, (((1,), (0,)), ((), ())),
            preferred_element_type=jnp.float32,
            precision=jax.lax.Precision.HIGHEST)
        sumsq_h = jnp.sum(wg * w1_ref[...], axis=1, keepdims=True)
        inv_count = 1.0 / float(n_batch * x_ref.shape[2])
        mean = sum_h * inv_count
        var = jnp.maximum(sumsq_h * inv_count - mean * mean, 0.0)
        scale = gamma_ref[...] * jax.lax.rsqrt(var + _BN_EPS)
        w1s_ref[...] = scale * w1_ref[...]
        shift_ref[...] = beta_ref[...] - mean * scale

    @pl.when(s > 0)
    def _apply():
        base = (s - 1) * apply_block
        for i in range(apply_block):
            xi = x_ref[base + i]
            h = jnp.dot(w1s_ref[...], xi,
                        preferred_element_type=jnp.float32)
            h = jnp.maximum(h + shift_ref[...], 0.0)
            out = jnp.dot(w2_ref[...], h,
                          preferred_element_type=jnp.float32) + b2_ref[...]
            o_ref[i] = out.astype(o_ref.dtype)


def kernel(x_nchw, w1, b1, gamma, beta, w2, b2):
    del b1
    N, Cin, H, W = x_nchw.shape
    Cinner = w1.shape[0]
    Cout = w2.shape[0]
    HW = H * W
    x3d = x_nchw.reshape(N, Cin, HW)

    apply_block = next(b for b in (4, 2, 1) if N % b == 0)
    n_apply = N // apply_block

    out3d = pl.pallas_call(
        functools.partial(_fused_kernel, n_batch=N, apply_block=apply_block),
        grid=(1 + n_apply,),
        in_specs=[
            pl.BlockSpec(memory_space=pltpu.VMEM),
            pl.BlockSpec(memory_space=pltpu.VMEM),
            pl.BlockSpec(memory_space=pltpu.VMEM),
            pl.BlockSpec(memory_space=pltpu.VMEM),
            pl.BlockSpec(memory_space=pltpu.VMEM),
            pl.BlockSpec(memory_space=pltpu.VMEM),
        ],
        out_specs=pl.BlockSpec(
            (apply_block, Cout, HW),
            lambda s: (jnp.maximum(s - 1, 0), 0, 0)),
        out_shape=jax.ShapeDtypeStruct((N, Cout, HW), x_nchw.dtype),
        scratch_shapes=[
            pltpu.VMEM((Cinner, Cin), jnp.float32),
            pltpu.VMEM((Cinner, 1), jnp.float32),
        ],
        compiler_params=pltpu.CompilerParams(
            dimension_semantics=("arbitrary",),
            vmem_limit_bytes=60 * 1024 * 1024,
        ),
        name="mlp2d_fused",
    )(x3d, w1, gamma, beta, w2, b2)

    return out3d.reshape(N, Cout, H, W)
